# baseline (speedup 1.0000x reference)
import functools

import jax
import jax.numpy as jnp
from jax import lax
from jax.experimental import pallas as pl
from jax.experimental.pallas import tpu as pltpu
from jax.experimental.pallas import tpu_sc as plsc

DIM = 64
BATCH = 16384
NC = 2
NS = 16
NW = NC * NS
CPW = BATCH // NW
CHUNK = 128
NCHUNK = CPW // CHUNK
LANES = 16
SROW = 2 * DIM


def _split_idx(src_v, samp_v, par_v):
    for i in range(CPW // LANES):
        sl = pl.ds(i * LANES, LANES)
        v = src_v[sl]
        samp_v[sl] = lax.shift_right_logical(v, 1)
        par_v[sl] = lax.shift_left(v & 1, 6)


def _score_chunk(es_v, ep_v, eo_v, pa_v, pb_v, pc_v, cbase, out_v):
    iota = lax.iota(jnp.int32, LANES)
    for g in range(CHUNK // LANES):
        off = pl.ds(cbase + g * LANES, LANES)
        rowv = iota + (g * LANES)
        ca0 = pa_v[off]
        cb0 = pb_v[off]
        cc0 = pc_v[off]

        def body(d, acc):
            a = plsc.load_gather(es_v, [rowv, ca0 + d])
            b = plsc.load_gather(ep_v, [rowv, cb0 + d])
            c = plsc.load_gather(eo_v, [rowv, cc0 + d])
            return acc + a * b * c

        acc = lax.fori_loop(0, DIM, body, jnp.zeros((LANES,), jnp.float32))
        out_v[pl.ds(cbase + g * LANES, LANES)] = acc


def _body(human2, gmf2, hm, rm, tm, hf, rf, tf,
          neg_o, dm_o, df_o,
          hmv, rmv, tmv, hfv, rfv, tfv,
          hs, rs, ts, hp, rp, tp,
          es_v, ep_v, eo_v,
          dm_v, df_v, ng_v,
          sem):
    wid = lax.axis_index("s") * NC + lax.axis_index("c")
    base = pl.multiple_of(wid * CPW, CPW)

    pltpu.sync_copy(hm.at[pl.ds(base, CPW)], hmv)
    pltpu.sync_copy(rm.at[pl.ds(base, CPW)], rmv)
    pltpu.sync_copy(tm.at[pl.ds(base, CPW)], tmv)
    pltpu.sync_copy(hf.at[pl.ds(base, CPW)], hfv)
    pltpu.sync_copy(rf.at[pl.ds(base, CPW)], rfv)
    pltpu.sync_copy(tf.at[pl.ds(base, CPW)], tfv)

    for (hv, rv, tv, out_v) in ((hmv, rmv, tmv, dm_v), (hfv, rfv, tfv, df_v)):
        _split_idx(hv, hs, hp)
        _split_idx(rv, rs, rp)
        _split_idx(tv, ts, tp)
        for c in range(NCHUNK):
            sl = pl.ds(c * CHUNK, CHUNK)
            cp1 = pltpu.make_async_copy(human2.at[hs.at[sl]], es_v, sem)
            cp2 = pltpu.make_async_copy(gmf2.at[rs.at[sl]], ep_v, sem)
            cp3 = pltpu.make_async_copy(gmf2.at[ts.at[sl]], eo_v, sem)
            cp1.start()
            cp2.start()
            cp3.start()
            cp1.wait()
            cp2.wait()
            cp3.wait()
            _score_chunk(es_v, ep_v, eo_v, hp, rp, tp, c * CHUNK, out_v)

    for i in range(CPW // LANES):
        sl = pl.ds(i * LANES, LANES)
        ng_v[sl] = df_v[sl] - dm_v[sl]

    pltpu.sync_copy(dm_v, dm_o.at[pl.ds(base, CPW)])
    pltpu.sync_copy(df_v, df_o.at[pl.ds(base, CPW)])
    pltpu.sync_copy(ng_v, neg_o.at[pl.ds(base, CPW)])


@jax.jit
def _run(human2, gmf2, hm, rm, tm, hf, rf, tf):
    out = jax.ShapeDtypeStruct((BATCH,), jnp.float32)
    idx_t = pltpu.VMEM((CPW,), jnp.int32)
    row_t = pltpu.VMEM((CHUNK, SROW), jnp.float32)
    res_t = pltpu.VMEM((CPW,), jnp.float32)
    k = functools.partial(
        pl.kernel,
        out_type=[out, out, out],
        mesh=plsc.VectorSubcoreMesh(core_axis_name="c", subcore_axis_name="s"),
        compiler_params=pltpu.CompilerParams(needs_layout_passes=False),
        scratch_types=[
            idx_t, idx_t, idx_t, idx_t, idx_t, idx_t,
            idx_t, idx_t, idx_t, idx_t, idx_t, idx_t,
            row_t, row_t, row_t,
            res_t, res_t, res_t,
            pltpu.SemaphoreType.DMA,
        ],
    )(_body)
    return k(human2, gmf2, hm, rm, tm, hf, rf, tf)


def kernel(human_embeds, gmf_embeds, male_triplets, female_triplets):
    human2 = human_embeds.reshape(-1, SROW)
    gmf2 = gmf_embeds.reshape(-1, SROW)
    hm = male_triplets[:, 0]
    rm = male_triplets[:, 1]
    tm = male_triplets[:, 2]
    hf = female_triplets[:, 0]
    rf = female_triplets[:, 1]
    tf = female_triplets[:, 2]
    neg, dm, df = _run(human2, gmf2, hm, rm, tm, hf, rf, tf)
    return (neg, dm, df)
